# baseline (speedup 1.0000x reference)
import functools

import jax
import jax.numpy as jnp
from jax.experimental import pallas as pl
from jax.experimental.pallas import tpu as pltpu


def _fused_body(w_ref, z_ref, o_ref):
    z = z_ref[0]
    k = z.shape[1]
    m = jnp.max(z, axis=1, keepdims=True)
    iota = jax.lax.broadcasted_iota(jnp.int32, z.shape, 1)
    idx = jnp.min(jnp.where(z == m, iota, k), axis=1, keepdims=True)
    onehot = (iota == idx).astype(jnp.bfloat16)
    o_ref[0] = jax.lax.dot_general(
        onehot, w_ref[...].astype(jnp.bfloat16), (((1,), (0,)), ((), ())),
        preferred_element_type=jnp.float32,
    )


@jax.jit
def kernel(z_e_x, weight):
    b, k, h, w = z_e_x.shape
    d = weight.shape[1]
    hw = h * w
    g = 4 if b % 4 == 0 else (2 if b % 2 == 0 else 1)
    z = jnp.transpose(z_e_x, (0, 2, 3, 1)).reshape(b // g, hw * g, k)
    out = pl.pallas_call(
        _fused_body,
        grid=(b // g,),
        in_specs=[
            pl.BlockSpec((k, d), lambda i: (0, 0)),
            pl.BlockSpec((1, hw * g, k), lambda i: (i, 0, 0)),
        ],
        out_specs=pl.BlockSpec((1, hw * g, d), lambda i: (i, 0, 0)),
        out_shape=jax.ShapeDtypeStruct((b // g, hw * g, d), jnp.float32),
    )(weight, z)
    return out.reshape(b, h, w, d).transpose(0, 3, 1, 2)
